# Initial kernel scaffold; baseline (speedup 1.0000x reference)
import jax
import jax.numpy as jnp
from jax.experimental import pallas as pl


def kernel(unary, binary, index1, index2):
    raise NotImplementedError("write your pallas kernel here")


# speedup vs baseline: 2.2702x; 2.2702x over previous
import functools

import jax
import jax.numpy as jnp
from jax import lax
from jax.experimental import pallas as pl
from jax.experimental.pallas import tpu as pltpu
from jax.experimental.pallas import tpu_sc as plsc

B = 320000
D = 128
DB = 16
NW = 32
SLAB = B // NW
C = 80
NCHUNK = SLAB // C


def _join_body(unary_hbm, binary_hbm, idx1_hbm, idx2_hbm, out_hbm,
               idx1_v, idx2_v, rows1_v, rows2_v, bin_v, sem):
    wid = lax.axis_index("s") * 2 + lax.axis_index("c")
    slab = wid * SLAB

    def step(j, carry):
        base = slab + j * C
        pltpu.sync_copy(idx1_hbm.at[pl.ds(base, C)], idx1_v)
        pltpu.sync_copy(idx2_hbm.at[pl.ds(base, C)], idx2_v)
        g1 = pltpu.async_copy(unary_hbm.at[idx1_v], rows1_v, sem)
        g2 = pltpu.async_copy(unary_hbm.at[idx2_v], rows2_v, sem)
        pltpu.sync_copy(binary_hbm.at[pl.ds(base, C)], bin_v)
        g1.wait()
        g2.wait()
        pltpu.sync_copy(rows1_v, out_hbm.at[pl.ds(base, C), pl.ds(0, D)])
        pltpu.sync_copy(rows2_v, out_hbm.at[pl.ds(base, C), pl.ds(D, D)])
        pltpu.sync_copy(bin_v, out_hbm.at[pl.ds(base, C), pl.ds(2 * D, DB)])
        return carry

    lax.fori_loop(0, NCHUNK, step, 0)


@jax.jit
def _join(unary, binary, index1, index2):
    mesh = plsc.VectorSubcoreMesh(core_axis_name="c", subcore_axis_name="s")
    return pl.kernel(
        _join_body,
        mesh=mesh,
        out_type=jax.ShapeDtypeStruct((B, 2 * D + DB), jnp.float32),
        scratch_types=[
            pltpu.VMEM((C,), jnp.int32),
            pltpu.VMEM((C,), jnp.int32),
            pltpu.VMEM((C, D), jnp.float32),
            pltpu.VMEM((C, D), jnp.float32),
            pltpu.VMEM((C, DB), jnp.float32),
            pltpu.SemaphoreType.DMA,
        ],
    )(unary, binary, index1, index2)


def kernel(unary, binary, index1, index2):
    index1 = jnp.squeeze(index1).astype(jnp.int32)
    index2 = jnp.squeeze(index2).astype(jnp.int32)
    return _join(unary, binary, index1, index2)


# speedup vs baseline: 2.8138x; 1.2395x over previous
import jax
import jax.numpy as jnp
from jax import lax
from jax.experimental import pallas as pl
from jax.experimental.pallas import tpu as pltpu
from jax.experimental.pallas import tpu_sc as plsc

B = 320000
D = 128
DB = 16
NW = 32
SLAB = B // NW
C = 40
NCHUNK = SLAB // C
NB = 4


def _join_body(unary_hbm, binary_hbm, idx1_hbm, idx2_hbm, out_hbm,
               idx1_v, idx2_v, rows1_v, rows2_v, bin_v, gsem, ssem):
    wid = lax.axis_index("s") * 2 + lax.axis_index("c")
    slab = wid * SLAB
    pltpu.sync_copy(idx1_hbm.at[pl.ds(slab, SLAB)], idx1_v)
    pltpu.sync_copy(idx2_hbm.at[pl.ds(slab, SLAB)], idx2_v)

    def g_copies(j, b):
        off = j * C
        sem = gsem.at[b]
        return (
            pltpu.make_async_copy(unary_hbm.at[idx1_v.at[pl.ds(off, C)]],
                                  rows1_v.at[b], sem),
            pltpu.make_async_copy(unary_hbm.at[idx2_v.at[pl.ds(off, C)]],
                                  rows2_v.at[b], sem),
            pltpu.make_async_copy(binary_hbm.at[pl.ds(slab + off, C)],
                                  bin_v.at[b], sem),
        )

    def s_copies(j, b):
        base = slab + j * C
        sem = ssem.at[b]
        return (
            pltpu.make_async_copy(rows1_v.at[b],
                                  out_hbm.at[pl.ds(base, C), pl.ds(0, D)], sem),
            pltpu.make_async_copy(rows2_v.at[b],
                                  out_hbm.at[pl.ds(base, C), pl.ds(D, D)], sem),
            pltpu.make_async_copy(bin_v.at[b],
                                  out_hbm.at[pl.ds(base, C), pl.ds(2 * D, DB)], sem),
        )

    def fire(copies):
        for c in copies:
            c.start()

    def drain(copies):
        for c in copies:
            c.wait()

    fire(g_copies(0, 0))
    fire(g_copies(1, 1))

    def step(j, carry):
        b = lax.rem(j, NB)
        drain(g_copies(j, b))
        fire(s_copies(j, b))

        @pl.when(j >= 2)
        def _():
            drain(s_copies(j - 2, lax.rem(j - 2, NB)))

        @pl.when(j + 2 < NCHUNK)
        def _():
            fire(g_copies(j + 2, lax.rem(j + 2, NB)))
        return carry

    lax.fori_loop(0, NCHUNK, step, 0)
    drain(s_copies(NCHUNK - 2, lax.rem(NCHUNK - 2, NB)))
    drain(s_copies(NCHUNK - 1, lax.rem(NCHUNK - 1, NB)))


@jax.jit
def _join(unary, binary, index1, index2):
    mesh = plsc.VectorSubcoreMesh(core_axis_name="c", subcore_axis_name="s")
    return pl.kernel(
        _join_body,
        mesh=mesh,
        out_type=jax.ShapeDtypeStruct((B, 2 * D + DB), jnp.float32),
        scratch_types=[
            pltpu.VMEM((SLAB,), jnp.int32),
            pltpu.VMEM((SLAB,), jnp.int32),
            pltpu.VMEM((NB, C, D), jnp.float32),
            pltpu.VMEM((NB, C, D), jnp.float32),
            pltpu.VMEM((NB, C, DB), jnp.float32),
            pltpu.SemaphoreType.DMA((NB,)),
            pltpu.SemaphoreType.DMA((NB,)),
        ],
    )(unary, binary, index1, index2)


def kernel(unary, binary, index1, index2):
    index1 = jnp.squeeze(index1).astype(jnp.int32)
    index2 = jnp.squeeze(index2).astype(jnp.int32)
    return _join(unary, binary, index1, index2)


# speedup vs baseline: 2.8235x; 1.0035x over previous
import jax
import jax.numpy as jnp
from jax import lax
from jax.experimental import pallas as pl
from jax.experimental.pallas import tpu as pltpu
from jax.experimental.pallas import tpu_sc as plsc

B = 320000
D = 128
DB = 16
NW = 32
SLAB = B // NW
C = 40
NCHUNK = SLAB // C
NB = 6
PF = 3
SD = NB - PF


def _join_body(unary_hbm, binary_hbm, idx1_hbm, idx2_hbm, out_hbm,
               idx1_v, idx2_v, rows1_v, rows2_v, bin_v, gsem, ssem):
    wid = lax.axis_index("s") * 2 + lax.axis_index("c")
    slab = wid * SLAB
    pltpu.sync_copy(idx1_hbm.at[pl.ds(slab, SLAB)], idx1_v)
    pltpu.sync_copy(idx2_hbm.at[pl.ds(slab, SLAB)], idx2_v)

    def g_copies(j, b):
        off = j * C
        sem = gsem.at[b]
        return (
            pltpu.make_async_copy(unary_hbm.at[idx1_v.at[pl.ds(off, C)]],
                                  rows1_v.at[b], sem),
            pltpu.make_async_copy(unary_hbm.at[idx2_v.at[pl.ds(off, C)]],
                                  rows2_v.at[b], sem),
            pltpu.make_async_copy(binary_hbm.at[pl.ds(slab + off, C)],
                                  bin_v.at[b], sem),
        )

    def s_copies(j, b):
        base = slab + j * C
        sem = ssem.at[b]
        return (
            pltpu.make_async_copy(rows1_v.at[b],
                                  out_hbm.at[pl.ds(base, C), pl.ds(0, D)], sem),
            pltpu.make_async_copy(rows2_v.at[b],
                                  out_hbm.at[pl.ds(base, C), pl.ds(D, D)], sem),
            pltpu.make_async_copy(bin_v.at[b],
                                  out_hbm.at[pl.ds(base, C), pl.ds(2 * D, DB)], sem),
        )

    def fire(copies):
        for c in copies:
            c.start()

    def drain(copies):
        for c in copies:
            c.wait()

    for j0 in range(PF):
        fire(g_copies(j0, j0))

    def step(j, carry):
        b = lax.rem(j, NB)
        drain(g_copies(j, b))
        fire(s_copies(j, b))

        @pl.when(j >= SD)
        def _():
            drain(s_copies(j - SD, lax.rem(j - SD, NB)))

        @pl.when(j + PF < NCHUNK)
        def _():
            fire(g_copies(j + PF, lax.rem(j + PF, NB)))
        return carry

    lax.fori_loop(0, NCHUNK, step, 0)
    for j0 in range(NCHUNK - SD, NCHUNK):
        drain(s_copies(j0, j0 % NB))


@jax.jit
def _join(unary, binary, index1, index2):
    mesh = plsc.VectorSubcoreMesh(core_axis_name="c", subcore_axis_name="s")
    return pl.kernel(
        _join_body,
        mesh=mesh,
        out_type=jax.ShapeDtypeStruct((B, 2 * D + DB), jnp.float32),
        scratch_types=[
            pltpu.VMEM((SLAB,), jnp.int32),
            pltpu.VMEM((SLAB,), jnp.int32),
            pltpu.VMEM((NB, C, D), jnp.float32),
            pltpu.VMEM((NB, C, D), jnp.float32),
            pltpu.VMEM((NB, C, DB), jnp.float32),
            pltpu.SemaphoreType.DMA((NB,)),
            pltpu.SemaphoreType.DMA((NB,)),
        ],
    )(unary, binary, index1, index2)


def kernel(unary, binary, index1, index2):
    index1 = jnp.squeeze(index1).astype(jnp.int32)
    index2 = jnp.squeeze(index2).astype(jnp.int32)
    return _join(unary, binary, index1, index2)


# speedup vs baseline: 9.0269x; 3.1970x over previous
import jax
import jax.numpy as jnp
from jax import lax
from jax.experimental import pallas as pl
from jax.experimental.pallas import tpu as pltpu
from jax.experimental.pallas import tpu_sc as plsc

B = 320000
D = 128
DB = 16
V = 10000
EC = 1280
NCH = B // EC
GPC = EC // 16
BE = 640
NBC = B // BE
NSP = 16
NKMAX = -(-NBC // NSP)


def _join_t_body(ut_hbm, bt_hbm, idx12_hbm, out_hbm,
                 urow_v, iva, ivb, bva, bvb, bn0, bn1, bn2, bn3,
                 gsa, gsb, ssa, ssb):
    wid = lax.axis_index("s") * 2 + lax.axis_index("c")
    g = wid // 16
    cbase = lax.rem(wid, 16) * 8
    rowbase = g * D + cbase
    ibase = g * B

    pltpu.sync_copy(ut_hbm.at[pl.ds(cbase, 8)], urow_v)

    def idx_copy(chunk, iv, sem):
        return pltpu.make_async_copy(
            idx12_hbm.at[pl.ds(ibase + chunk * EC, EC)], iv, sem)

    def store_copy(chunk, bv, sem):
        base = chunk * EC
        return pltpu.make_async_copy(
            bv, out_hbm.at[pl.ds(rowbase, 8), pl.ds(base, EC)], sem)

    vcs = [jnp.full((16,), c, dtype=jnp.int32) for c in range(8)]

    def compute(iv, bv):
        @plsc.parallel_loop(0, EC, step=16, unroll=4)
        def _(off):
            vi = iv[pl.ds(off, 16)]
            for c in range(8):
                bv[c, pl.ds(off, 16)] = plsc.load_gather(urow_v, [vcs[c], vi])

    idx_copy(0, iva, gsa).start()

    def step(t, carry):
        ca = 2 * t
        cb = 2 * t + 1
        idx_copy(cb, ivb, gsb).start()
        idx_copy(ca, iva, gsa).wait()

        @pl.when(t >= 1)
        def _():
            store_copy(ca - 2, bva, ssa).wait()
        compute(iva, bva)
        store_copy(ca, bva, ssa).start()

        @pl.when(cb + 1 < NCH)
        def _():
            idx_copy(cb + 1, iva, gsa).start()
        idx_copy(cb, ivb, gsb).wait()

        @pl.when(t >= 1)
        def _():
            store_copy(cb - 2, bvb, ssb).wait()
        compute(ivb, bvb)
        store_copy(cb, bvb, ssb).start()
        return carry

    lax.fori_loop(0, NCH // 2, step, 0)
    store_copy(NCH - 2, bva, ssa).wait()
    store_copy(NCH - 1, bvb, ssb).wait()

    bb = lax.rem(wid, 2) * 8
    sp = wid // 2
    bbufs = (bn0, bn1, bn2, bn3)
    lsems = (gsa, gsb)
    tsems = (ssa, ssb)

    def bchunk(k):
        return sp + NSP * k

    def bin_load(k, slot):
        base = bchunk(k) * BE
        return pltpu.make_async_copy(
            bt_hbm.at[pl.ds(bb, 8), pl.ds(base, BE)],
            bbufs[slot], lsems[slot % 2])

    def bin_store(k, slot):
        base = bchunk(k) * BE
        return pltpu.make_async_copy(
            bbufs[slot], out_hbm.at[pl.ds(2 * D + bb, 8), pl.ds(base, BE)],
            tsems[slot % 2])

    bin_load(0, 0).start()
    bin_load(1, 1).start()

    def bstep(t, carry):
        for u in range(4):
            k = 4 * t + u

            @pl.when(bchunk(k) < NBC)
            def _():
                bin_load(k, u).wait()
                bin_store(k, u).start()

            @pl.when((k >= 2) & (bchunk(k - 2) < NBC))
            def _():
                bin_store(k - 2, (u + 2) % 4).wait()

            @pl.when(bchunk(k + 2) < NBC)
            def _():
                bin_load(k + 2, (u + 2) % 4).start()
        return carry

    lax.fori_loop(0, -(-NKMAX // 4), bstep, 0)
    kk = NKMAX - 1
    @pl.when(bchunk(kk) < NBC)
    def _():
        bin_store(kk, kk % 4).wait()


@jax.jit
def _join(unary, binary, index1, index2):
    mesh = plsc.VectorSubcoreMesh(core_axis_name="c", subcore_axis_name="s")
    out_t = pl.kernel(
        _join_t_body,
        mesh=mesh,
        compiler_params=pltpu.CompilerParams(needs_layout_passes=False),
        out_type=jax.ShapeDtypeStruct((2 * D + DB, B), jnp.float32),
        scratch_types=[
            pltpu.VMEM((8, V), jnp.float32),
            pltpu.VMEM((EC,), jnp.int32),
            pltpu.VMEM((EC,), jnp.int32),
            pltpu.VMEM((8, EC), jnp.float32),
            pltpu.VMEM((8, EC), jnp.float32),
            pltpu.VMEM((8, BE), jnp.float32),
            pltpu.VMEM((8, BE), jnp.float32),
            pltpu.VMEM((8, BE), jnp.float32),
            pltpu.VMEM((8, BE), jnp.float32),
            pltpu.SemaphoreType.DMA,
            pltpu.SemaphoreType.DMA,
            pltpu.SemaphoreType.DMA,
            pltpu.SemaphoreType.DMA,
        ],
    )(jnp.transpose(unary), jnp.transpose(binary),
      jnp.concatenate([index1, index2]))
    return jnp.transpose(out_t)


def kernel(unary, binary, index1, index2):
    index1 = jnp.squeeze(index1).astype(jnp.int32)
    index2 = jnp.squeeze(index2).astype(jnp.int32)
    return _join(unary, binary, index1, index2)
